# baseline (speedup 1.0000x reference)
import jax
import jax.numpy as jnp
from jax import lax
from jax.experimental import pallas as pl
from jax.experimental.pallas import tpu as pltpu

IN_FEATURES = 512
OUT_FEATURES = 512
NUM_CONNECTIONS = 32
BATCH_BLOCK = 512
_ROW_CHUNK = 64


def _agg_kernel(conn_ref, op_ref, x_ref, fwd_ref, out_ref, m_ref):
    step = pl.program_id(0)

    @pl.when(step == 0)
    def _build_m():
        iota_i = lax.broadcasted_iota(
            jnp.int32, (_ROW_CHUNK, OUT_FEATURES), 0
        )
        for chunk in range(IN_FEATURES // _ROW_CHUNK):
            iota_c = iota_i + (chunk * _ROW_CHUNK)
            acc = jnp.zeros((_ROW_CHUNK, OUT_FEATURES), jnp.float32)
            for c in range(NUM_CONNECTIONS):
                row = conn_ref[pl.ds(c, 1), :]
                acc = acc + (iota_c == row).astype(jnp.float32)
            m_ref[pl.ds(chunk * _ROW_CHUNK, _ROW_CHUNK), :] = acc

    s = jnp.dot(x_ref[...], m_ref[...], preferred_element_type=jnp.float32)
    mean = s * (1.0 / NUM_CONNECTIONS)
    opi = op_ref[0, :]
    fwd_ref[...] = jnp.where((opi == 0)[None, :], s, mean)
    out_ref[:, 0, :] = s
    out_ref[:, 1, :] = mean


@jax.jit
def kernel(x, connection_indices, operator_table_indices):
    batch = x.shape[0]
    conn_t = connection_indices.T
    op_row = operator_table_indices.reshape(1, OUT_FEATURES)
    grid = (batch // BATCH_BLOCK,)
    fwd, out = pl.pallas_call(
        _agg_kernel,
        grid=grid,
        in_specs=[
            pl.BlockSpec((NUM_CONNECTIONS, OUT_FEATURES), lambda i: (0, 0)),
            pl.BlockSpec((1, OUT_FEATURES), lambda i: (0, 0)),
            pl.BlockSpec((BATCH_BLOCK, IN_FEATURES), lambda i: (i, 0)),
        ],
        out_specs=[
            pl.BlockSpec((BATCH_BLOCK, OUT_FEATURES), lambda i: (i, 0)),
            pl.BlockSpec((BATCH_BLOCK, 2, OUT_FEATURES), lambda i: (i, 0, 0)),
        ],
        out_shape=[
            jax.ShapeDtypeStruct((batch, OUT_FEATURES), jnp.float32),
            jax.ShapeDtypeStruct((batch, 2, OUT_FEATURES), jnp.float32),
        ],
        scratch_shapes=[pltpu.VMEM((IN_FEATURES, OUT_FEATURES), jnp.float32)],
        compiler_params=pltpu.CompilerParams(
            dimension_semantics=("arbitrary",),
        ),
    )(conn_t, op_row, x)
    return (fwd, out)


# speedup vs baseline: 1.0757x; 1.0757x over previous
import jax
import jax.numpy as jnp
from jax import lax
from jax.experimental import pallas as pl
from jax.experimental.pallas import tpu as pltpu

IN_FEATURES = 512
OUT_FEATURES = 512
NUM_CONNECTIONS = 32
BATCH_BLOCK = 2048
_ROW_CHUNK = 64


def _agg_kernel(conn_ref, op_ref, x_ref, fwd_ref, out_ref, m_ref):
    step = pl.program_id(0)

    @pl.when(step == 0)
    def _build_m():
        iota_i = lax.broadcasted_iota(
            jnp.int32, (_ROW_CHUNK, OUT_FEATURES), 0
        )
        for chunk in range(IN_FEATURES // _ROW_CHUNK):
            iota_c = iota_i + (chunk * _ROW_CHUNK)
            acc = jnp.zeros((_ROW_CHUNK, OUT_FEATURES), jnp.float32)
            for c in range(NUM_CONNECTIONS):
                row = conn_ref[pl.ds(c, 1), :]
                acc = acc + (iota_c == row).astype(jnp.float32)
            m_ref[pl.ds(chunk * _ROW_CHUNK, _ROW_CHUNK), :] = acc

    s = jnp.dot(x_ref[...], m_ref[...], preferred_element_type=jnp.float32)
    mean = s * (1.0 / NUM_CONNECTIONS)
    opi = op_ref[0, :]
    fwd_ref[...] = jnp.where((opi == 0)[None, :], s, mean)
    out_ref[:, 0, :] = s
    out_ref[:, 1, :] = mean


@jax.jit
def kernel(x, connection_indices, operator_table_indices):
    batch = x.shape[0]
    conn_t = connection_indices.T
    op_row = operator_table_indices.reshape(1, OUT_FEATURES)
    grid = (batch // BATCH_BLOCK,)
    fwd, out = pl.pallas_call(
        _agg_kernel,
        grid=grid,
        in_specs=[
            pl.BlockSpec((NUM_CONNECTIONS, OUT_FEATURES), lambda i: (0, 0)),
            pl.BlockSpec((1, OUT_FEATURES), lambda i: (0, 0)),
            pl.BlockSpec((BATCH_BLOCK, IN_FEATURES), lambda i: (i, 0)),
        ],
        out_specs=[
            pl.BlockSpec((BATCH_BLOCK, OUT_FEATURES), lambda i: (i, 0)),
            pl.BlockSpec((BATCH_BLOCK, 2, OUT_FEATURES), lambda i: (i, 0, 0)),
        ],
        out_shape=[
            jax.ShapeDtypeStruct((batch, OUT_FEATURES), jnp.float32),
            jax.ShapeDtypeStruct((batch, 2, OUT_FEATURES), jnp.float32),
        ],
        scratch_shapes=[pltpu.VMEM((IN_FEATURES, OUT_FEATURES), jnp.float32)],
        compiler_params=pltpu.CompilerParams(
            dimension_semantics=("arbitrary",),
        ),
    )(conn_t, op_row, x)
    return (fwd, out)


# speedup vs baseline: 1.1689x; 1.0866x over previous
import jax
import jax.numpy as jnp
from jax import lax
from jax.experimental import pallas as pl
from jax.experimental.pallas import tpu as pltpu

IN_FEATURES = 512
OUT_FEATURES = 512
NUM_CONNECTIONS = 32
BATCH_BLOCK = 2048
_ROW_CHUNK = 64


def _agg_kernel(conn_ref, op_ref, x_ref, fwd_ref, out_ref, m_ref):
    step = pl.program_id(0)

    @pl.when(step == 0)
    def _build_m():
        conn16 = conn_ref[...].astype(jnp.int16)
        iota_i = lax.broadcasted_iota(
            jnp.int16, (_ROW_CHUNK, OUT_FEATURES), 0
        )
        for chunk in range(IN_FEATURES // _ROW_CHUNK):
            iota_c = iota_i + jnp.int16(chunk * _ROW_CHUNK)
            acc = jnp.zeros((_ROW_CHUNK, OUT_FEATURES), jnp.int16)
            for c in range(NUM_CONNECTIONS):
                row = lax.slice(conn16, (c, 0), (c + 1, OUT_FEATURES))
                acc = acc + (iota_c == row).astype(jnp.int16)
            m_ref[pl.ds(chunk * _ROW_CHUNK, _ROW_CHUNK), :] = acc.astype(
                jnp.float32)

    s = jnp.dot(x_ref[...], m_ref[...], preferred_element_type=jnp.float32)
    mean = s * (1.0 / NUM_CONNECTIONS)
    opi = op_ref[0, :]
    fwd_ref[...] = jnp.where((opi == 0)[None, :], s, mean)
    out_ref[:, 0, :] = s
    out_ref[:, 1, :] = mean


@jax.jit
def kernel(x, connection_indices, operator_table_indices):
    batch = x.shape[0]
    conn_t = connection_indices.T
    op_row = operator_table_indices.reshape(1, OUT_FEATURES)
    grid = (batch // BATCH_BLOCK,)
    fwd, out = pl.pallas_call(
        _agg_kernel,
        grid=grid,
        in_specs=[
            pl.BlockSpec((NUM_CONNECTIONS, OUT_FEATURES), lambda i: (0, 0)),
            pl.BlockSpec((1, OUT_FEATURES), lambda i: (0, 0)),
            pl.BlockSpec((BATCH_BLOCK, IN_FEATURES), lambda i: (i, 0)),
        ],
        out_specs=[
            pl.BlockSpec((BATCH_BLOCK, OUT_FEATURES), lambda i: (i, 0)),
            pl.BlockSpec((BATCH_BLOCK, 2, OUT_FEATURES), lambda i: (i, 0, 0)),
        ],
        out_shape=[
            jax.ShapeDtypeStruct((batch, OUT_FEATURES), jnp.float32),
            jax.ShapeDtypeStruct((batch, 2, OUT_FEATURES), jnp.float32),
        ],
        scratch_shapes=[pltpu.VMEM((IN_FEATURES, OUT_FEATURES), jnp.float32)],
        compiler_params=pltpu.CompilerParams(
            dimension_semantics=("arbitrary",),
        ),
    )(conn_t, op_row, x)
    return (fwd, out)


# speedup vs baseline: 1.1769x; 1.0068x over previous
import jax
import jax.numpy as jnp
from jax import lax
from jax.experimental import pallas as pl
from jax.experimental.pallas import tpu as pltpu

IN_FEATURES = 512
OUT_FEATURES = 512
NUM_CONNECTIONS = 32
BATCH_BLOCK = 1024
_ROW_CHUNK = 64


def _agg_kernel(conn_ref, op_ref, x_ref, fwd_ref, out_ref, m_ref):
    step = pl.program_id(0)

    @pl.when(step == 0)
    def _build_m():
        conn16 = conn_ref[...].astype(jnp.int16)
        iota_i = lax.broadcasted_iota(
            jnp.int16, (_ROW_CHUNK, OUT_FEATURES), 0
        )
        for chunk in range(IN_FEATURES // _ROW_CHUNK):
            iota_c = iota_i + jnp.int16(chunk * _ROW_CHUNK)
            acc = jnp.zeros((_ROW_CHUNK, OUT_FEATURES), jnp.int16)
            for c in range(NUM_CONNECTIONS):
                row = lax.slice(conn16, (c, 0), (c + 1, OUT_FEATURES))
                acc = acc + (iota_c == row).astype(jnp.int16)
            m_ref[pl.ds(chunk * _ROW_CHUNK, _ROW_CHUNK), :] = acc.astype(
                jnp.float32)

    s = jnp.dot(x_ref[...], m_ref[...], preferred_element_type=jnp.float32)
    mean = s * (1.0 / NUM_CONNECTIONS)
    opi = op_ref[0, :]
    fwd_ref[...] = jnp.where((opi == 0)[None, :], s, mean)
    out_ref[:, 0, :] = s
    out_ref[:, 1, :] = mean


@jax.jit
def kernel(x, connection_indices, operator_table_indices):
    batch = x.shape[0]
    conn_t = connection_indices.T
    op_row = operator_table_indices.reshape(1, OUT_FEATURES)
    grid = (batch // BATCH_BLOCK,)
    fwd, out = pl.pallas_call(
        _agg_kernel,
        grid=grid,
        in_specs=[
            pl.BlockSpec((NUM_CONNECTIONS, OUT_FEATURES), lambda i: (0, 0)),
            pl.BlockSpec((1, OUT_FEATURES), lambda i: (0, 0)),
            pl.BlockSpec((BATCH_BLOCK, IN_FEATURES), lambda i: (i, 0)),
        ],
        out_specs=[
            pl.BlockSpec((BATCH_BLOCK, OUT_FEATURES), lambda i: (i, 0)),
            pl.BlockSpec((BATCH_BLOCK, 2, OUT_FEATURES), lambda i: (i, 0, 0)),
        ],
        out_shape=[
            jax.ShapeDtypeStruct((batch, OUT_FEATURES), jnp.float32),
            jax.ShapeDtypeStruct((batch, 2, OUT_FEATURES), jnp.float32),
        ],
        scratch_shapes=[pltpu.VMEM((IN_FEATURES, OUT_FEATURES), jnp.float32)],
        compiler_params=pltpu.CompilerParams(
            dimension_semantics=("arbitrary",),
        ),
    )(conn_t, op_row, x)
    return (fwd, out)
